# baseline (speedup 1.0000x reference)
import jax
import jax.numpy as jnp
import numpy as np
from jax.experimental import pallas as pl
from jax.experimental.pallas import tpu as pltpu

_H_IMG, _W_IMG = 832.0, 1472.0
_NUM_CLASSES = 4
_LAMBDA_BOX = 5.0
_B, _N, _T = 256, 1196, 64

_G = 4


def _loss_kernel(pref, tref, oref):
    for g in range(_G):
        _one_batch(pref[g], tref[g], oref, g)


def _one_batch(p, tg, oref, g):

    cx = (p[0:1, :] * 2.0 - 1.0) * (_W_IMG / 2.0)
    cy = (p[1:2, :] * 2.0 - 1.0) * (_H_IMG / 2.0)
    bw = jnp.exp(p[2:3, :]) * 32.0
    bh = jnp.exp(p[3:4, :]) * 32.0
    bx1 = cx - bw / 2
    by1 = cy - bh / 2
    bx2 = cx + bw / 2
    by2 = cy + bh / 2

    tx1 = tg[:, 0:1]
    ty1 = tg[:, 1:2]
    tx2 = tg[:, 2:3]
    ty2 = tg[:, 3:4]
    tcls = tg[:, 4:5]

    iw = jnp.maximum(jnp.minimum(bx2, tx2) - jnp.maximum(bx1, tx1), 0.0)
    ih = jnp.maximum(jnp.minimum(by2, ty2) - jnp.maximum(by1, ty1), 0.0)
    inter = iw * ih
    a1 = (bx2 - bx1) * (by2 - by1)
    a2 = (tx2 - tx1) * (ty2 - ty1)
    union = a1 + a2 - inter
    iou = inter / union

    niota = jax.lax.broadcasted_iota(jnp.int32, (_T, _N), 1)
    m = jnp.max(iou, axis=1, keepdims=True)
    idx = jnp.min(jnp.where(iou == m, niota, _N), axis=1, keepdims=True)
    onehot = jnp.where(niota == idx, 1.0, 0.0)

    rows = jnp.concatenate(
        [bx1, by1, bx2, by2, p[5:6, :], p[6:7, :], p[7:8, :], p[8:9, :]],
        axis=0)
    gath = jax.lax.dot_general(
        onehot, rows, (((1,), (1,)), ((), ())),
        preferred_element_type=jnp.float32)

    pmx1 = gath[:, 0:1]
    pmy1 = gath[:, 1:2]
    pmx2 = gath[:, 2:3]
    pmy2 = gath[:, 3:4]
    l0 = gath[:, 4:5]
    l1 = gath[:, 5:6]
    l2 = gath[:, 6:7]
    l3 = gath[:, 7:8]

    def smooth_l1(dd):
        add = jnp.abs(dd)
        return jnp.where(add < 1.0, 0.5 * dd * dd, add - 0.5)

    box = (smooth_l1(pmx1 - tx1) + smooth_l1(pmy1 - ty1)
           + smooth_l1(pmx2 - tx2) + smooth_l1(pmy2 - ty2))

    mx = jnp.maximum(jnp.maximum(l0, l1), jnp.maximum(l2, l3))
    se = (jnp.exp(l0 - mx) + jnp.exp(l1 - mx)
          + jnp.exp(l2 - mx) + jnp.exp(l3 - mx))
    lse = jnp.log(se) + mx
    picked = jnp.where(tcls == 0.0, l0,
             jnp.where(tcls == 1.0, l1,
             jnp.where(tcls == 2.0, l2, l3)))
    cls = lse - picked

    pos = jnp.max(onehot, axis=0, keepdims=True)
    xc = p[4:5, :]
    bce = jnp.maximum(xc, 0.0) - xc * pos + jnp.log1p(jnp.exp(-jnp.abs(xc)))
    conf = jnp.sum(bce, axis=1, keepdims=True)

    per_t = _LAMBDA_BOX * box + cls
    tot = jnp.sum(per_t, axis=0, keepdims=True) + conf
    oref[g, :, :] = jnp.broadcast_to(tot, (1, 128))


def _shard_loss(pred_shard, targets_shard):
    bs = pred_shard.shape[0]
    pt = jnp.transpose(pred_shard, (0, 2, 1))
    return pl.pallas_call(
        _loss_kernel,
        grid=(bs // _G,),
        in_specs=[
            pl.BlockSpec((_G, 9, _N), lambda b: (b, 0, 0)),
            pl.BlockSpec((_G, _T, 5), lambda b: (b, 0, 0)),
        ],
        out_specs=pl.BlockSpec((_G, 1, 128), lambda b: (b, 0, 0)),
        out_shape=jax.ShapeDtypeStruct((bs, 1, 128), jnp.float32),
        compiler_params=pltpu.CompilerParams(
            dimension_semantics=("parallel",)),
    )(pt, targets_shard)


def kernel(predictions, targets):
    devs = jax.devices()
    nd = 2 if len(devs) >= 2 and _B % (2 * _G) == 0 else 1
    if nd > 1:
        mesh = jax.sharding.Mesh(np.array(devs[:nd]), ("d",))
        spec = jax.sharding.PartitionSpec("d")
        out = jax.shard_map(
            _shard_loss, mesh=mesh,
            in_specs=(spec, spec), out_specs=spec, check_vma=False,
        )(predictions, targets)
    else:
        out = _shard_loss(predictions, targets)
    return jnp.sum(out[:, 0, 0]) / _B


# speedup vs baseline: 2.8057x; 2.8057x over previous
import jax
import jax.numpy as jnp
import numpy as np
from jax.experimental import pallas as pl
from jax.experimental.pallas import tpu as pltpu

_H_IMG, _W_IMG = 832.0, 1472.0
_NUM_CLASSES = 4
_LAMBDA_BOX = 5.0
_B, _N, _T = 256, 1196, 64

_G = 4


def _loss_kernel(pref, tref, oref):
    for g in range(_G):
        _one_batch(pref[g], tref[g], oref, g)


def _one_batch(p, tg, oref, g):

    cx = (p[0:1, :] * 2.0 - 1.0) * (_W_IMG / 2.0)
    cy = (p[1:2, :] * 2.0 - 1.0) * (_H_IMG / 2.0)
    bw = jnp.exp(p[2:3, :]) * 32.0
    bh = jnp.exp(p[3:4, :]) * 32.0
    bx1 = cx - bw / 2
    by1 = cy - bh / 2
    bx2 = cx + bw / 2
    by2 = cy + bh / 2

    tx1 = tg[:, 0:1]
    ty1 = tg[:, 1:2]
    tx2 = tg[:, 2:3]
    ty2 = tg[:, 3:4]
    tcls = tg[:, 4:5]

    iw = jnp.maximum(jnp.minimum(bx2, tx2) - jnp.maximum(bx1, tx1), 0.0)
    ih = jnp.maximum(jnp.minimum(by2, ty2) - jnp.maximum(by1, ty1), 0.0)
    inter = iw * ih
    a1 = (bx2 - bx1) * (by2 - by1)
    a2 = (tx2 - tx1) * (ty2 - ty1)
    union = a1 + a2 - inter
    iou = inter / union

    niota = jax.lax.broadcasted_iota(jnp.int32, (_T, _N), 1)
    m = jnp.max(iou, axis=1, keepdims=True)
    idx = jnp.min(jnp.where(iou == m, niota, _N), axis=1, keepdims=True)
    onehot = jnp.where(niota == idx, 1.0, 0.0)

    rows = jnp.concatenate(
        [bx1, by1, bx2, by2, p[5:6, :], p[6:7, :], p[7:8, :], p[8:9, :]],
        axis=0)
    gath = jax.lax.dot_general(
        onehot, rows, (((1,), (1,)), ((), ())),
        preferred_element_type=jnp.float32)

    pmx1 = gath[:, 0:1]
    pmy1 = gath[:, 1:2]
    pmx2 = gath[:, 2:3]
    pmy2 = gath[:, 3:4]
    l0 = gath[:, 4:5]
    l1 = gath[:, 5:6]
    l2 = gath[:, 6:7]
    l3 = gath[:, 7:8]

    def smooth_l1(dd):
        add = jnp.abs(dd)
        return jnp.where(add < 1.0, 0.5 * dd * dd, add - 0.5)

    box = (smooth_l1(pmx1 - tx1) + smooth_l1(pmy1 - ty1)
           + smooth_l1(pmx2 - tx2) + smooth_l1(pmy2 - ty2))

    mx = jnp.maximum(jnp.maximum(l0, l1), jnp.maximum(l2, l3))
    se = (jnp.exp(l0 - mx) + jnp.exp(l1 - mx)
          + jnp.exp(l2 - mx) + jnp.exp(l3 - mx))
    lse = jnp.log(se) + mx
    picked = jnp.where(tcls == 0.0, l0,
             jnp.where(tcls == 1.0, l1,
             jnp.where(tcls == 2.0, l2, l3)))
    cls = lse - picked

    pos = jnp.max(onehot, axis=0, keepdims=True)
    xc = p[4:5, :]
    bce = jnp.maximum(xc, 0.0) - xc * pos + jnp.log1p(jnp.exp(-jnp.abs(xc)))
    conf = jnp.sum(bce, axis=1, keepdims=True)

    per_t = _LAMBDA_BOX * box + cls
    tot = jnp.sum(per_t, axis=0, keepdims=True) + conf
    oref[g, :, :] = jnp.broadcast_to(tot, (1, 128))


def _shard_loss(pred_shard, targets_shard):
    bs = pred_shard.shape[0]
    pt = jnp.transpose(pred_shard, (0, 2, 1))
    return pl.pallas_call(
        _loss_kernel,
        grid=(bs // _G,),
        in_specs=[
            pl.BlockSpec((_G, 9, _N), lambda b: (b, 0, 0)),
            pl.BlockSpec((_G, _T, 5), lambda b: (b, 0, 0)),
        ],
        out_specs=pl.BlockSpec((_G, 1, 128), lambda b: (b, 0, 0)),
        out_shape=jax.ShapeDtypeStruct((bs, 1, 128), jnp.float32),
        compiler_params=pltpu.CompilerParams(
            dimension_semantics=("parallel",),
            allow_input_fusion=[True, False]),
    )(pt, targets_shard)


def kernel(predictions, targets):
    out = _shard_loss(predictions, targets)
    return jnp.sum(out[:, 0, 0]) / _B
